# Initial kernel scaffold; baseline (speedup 1.0000x reference)
import jax
import jax.numpy as jnp
from jax.experimental import pallas as pl


def kernel(utt_embeds, speaker_ids, gW_ih, gW_hh, gb_ih, gb_hh, sW_ih, sW_hh, sb_ih, sb_hh, eW_ih, eW_hh, eb_ih, eb_hh):
    raise NotImplementedError("write your pallas kernel here")


# speedup vs baseline: 6.7560x; 6.7560x over previous
import jax
import jax.numpy as jnp
from jax.experimental import pallas as pl
from jax.experimental.pallas import tpu as pltpu


def _proj_body(u_ref, w_ref, b_ref, o_ref):
    o_ref[...] = (
        jnp.dot(u_ref[...], w_ref[...], preferred_element_type=jnp.float32)
        + b_ref[...]
    )


def _project(ut, wx, bx, interpret=False):
    M, D = ut.shape
    N = wx.shape[1]
    bm = min(1024, M)
    bn = min(1536, N)
    grid = (M // bm, N // bn)
    return pl.pallas_call(
        _proj_body,
        out_shape=jax.ShapeDtypeStruct((M, N), jnp.float32),
        grid=grid,
        in_specs=[
            pl.BlockSpec((bm, D), lambda i, j: (i, 0)),
            pl.BlockSpec((D, bn), lambda i, j: (0, j)),
            pl.BlockSpec((1, bn), lambda i, j: (0, j)),
        ],
        out_specs=pl.BlockSpec((bm, bn), lambda i, j: (i, j)),
        compiler_params=pltpu.CompilerParams(
            dimension_semantics=("parallel", "parallel"),
            vmem_limit_bytes=48 * 1024 * 1024,
        ),
        name="speaker_rnn_project",
        interpret=interpret,
    )(ut, wx, bx)


def _make_rnn_body(H):
    def _rnn_body(m_ref, xp_ref, wg_ref, wsg_ref, wsh_ref, wes_ref, bn_ref,
                  out_ref, g_ref, s0_ref, s1_ref):
        t = pl.program_id(1)

        @pl.when(t == 0)
        def _():
            g_ref[...] = jnp.zeros_like(g_ref)
            s0_ref[...] = jnp.zeros_like(s0_ref)
            s1_ref[...] = jnp.zeros_like(s1_ref)

        xp = xp_ref[0, 0]
        g = g_ref[...]

        hh = jnp.dot(g, wg_ref[...], preferred_element_type=jnp.float32)
        r = jax.nn.sigmoid(xp[:, :H] + hh[:, :H])
        z = jax.nn.sigmoid(xp[:, H:2 * H] + hh[:, H:2 * H])
        n = jnp.tanh(xp[:, 2 * H:3 * H] + r * (hh[:, 2 * H:] + bn_ref[0:1, :]))
        g_new = (1.0 - z) * n + z * g
        g_ref[...] = g_new

        m = m_ref[0, 0][:, :1]
        s0 = s0_ref[...]
        s1 = s1_ref[...]
        s_prev = jnp.where(m < 0.5, s0, s1)
        sg = jnp.dot(g_new, wsg_ref[...], preferred_element_type=jnp.float32)
        sh = jnp.dot(s_prev, wsh_ref[...], preferred_element_type=jnp.float32)
        ps = xp[:, 3 * H:6 * H] + sg
        r_s = jax.nn.sigmoid(ps[:, :H] + sh[:, :H])
        z_s = jax.nn.sigmoid(ps[:, H:2 * H] + sh[:, H:2 * H])
        n_s = jnp.tanh(ps[:, 2 * H:] + r_s * (sh[:, 2 * H:] + bn_ref[1:2, :]))
        s_new = (1.0 - z_s) * n_s + z_s * s_prev
        s0_ref[...] = jnp.where(m < 0.5, s_new, s0)
        s1_ref[...] = jnp.where(m < 0.5, s1, s_new)

        es = jnp.dot(s_new, wes_ref[...], preferred_element_type=jnp.float32)
        pe = xp[:, 6 * H:] + es
        r_e = jax.nn.sigmoid(pe[:, :H])
        z_e = jax.nn.sigmoid(pe[:, H:2 * H])
        n_e = jnp.tanh(pe[:, 2 * H:] + r_e * bn_ref[2:3, :])
        out_ref[0, 0] = (1.0 - z_e) * n_e

    return _rnn_body


def _forward(utt_embeds, speaker_ids,
             gW_ih, gW_hh, gb_ih, gb_hh,
             sW_ih, sW_hh, sb_ih, sb_hh,
             eW_ih, eW_hh, eb_ih, eb_hh,
             interpret=False):
    B, T, D = utt_embeds.shape
    H = gW_hh.shape[1]
    Bc = B // 2

    f32 = jnp.float32
    utt_embeds = utt_embeds.astype(f32)

    wx = jnp.concatenate([gW_ih, sW_ih[:, :D], eW_ih[:, :D]], axis=0).T

    def fold(b_ih, b_hh):
        return b_ih + jnp.concatenate([b_hh[:2 * H], jnp.zeros((H,), f32)])

    bx = jnp.concatenate(
        [fold(gb_ih, gb_hh), fold(sb_ih, sb_hh), fold(eb_ih, eb_hh)]
    ).reshape(1, 9 * H).astype(f32)

    ut = jnp.swapaxes(utt_embeds, 0, 1).reshape(T * B, D)
    xp = _project(ut, wx.astype(f32), bx, interpret=interpret)
    xp4 = xp.reshape(T, 2, Bc, 9 * H)

    mcol = jnp.broadcast_to(
        jnp.swapaxes(speaker_ids, 0, 1).astype(f32).reshape(T, 2, Bc, 1),
        (T, 2, Bc, 128),
    )

    wg = gW_hh.T.astype(f32)
    wsg = sW_ih[:, D:].T.astype(f32)
    wsh = sW_hh.T.astype(f32)
    wes = eW_ih[:, D:].T.astype(f32)
    bn = jnp.stack([gb_hh[2 * H:], sb_hh[2 * H:], eb_hh[2 * H:]]).astype(f32)

    out = pl.pallas_call(
        _make_rnn_body(H),
        out_shape=jax.ShapeDtypeStruct((T, 2, Bc, H), jnp.float32),
        grid=(2, T),
        in_specs=[
            pl.BlockSpec((1, 1, Bc, 128), lambda c, t: (t, c, 0, 0)),
            pl.BlockSpec((1, 1, Bc, 9 * H), lambda c, t: (t, c, 0, 0)),
            pl.BlockSpec((H, 3 * H), lambda c, t: (0, 0)),
            pl.BlockSpec((H, 3 * H), lambda c, t: (0, 0)),
            pl.BlockSpec((H, 3 * H), lambda c, t: (0, 0)),
            pl.BlockSpec((H, 3 * H), lambda c, t: (0, 0)),
            pl.BlockSpec((3, H), lambda c, t: (0, 0)),
        ],
        out_specs=pl.BlockSpec((1, 1, Bc, H), lambda c, t: (t, c, 0, 0)),
        scratch_shapes=[
            pltpu.VMEM((Bc, H), jnp.float32),
            pltpu.VMEM((Bc, H), jnp.float32),
            pltpu.VMEM((Bc, H), jnp.float32),
        ],
        compiler_params=pltpu.CompilerParams(
            dimension_semantics=("parallel", "arbitrary"),
            vmem_limit_bytes=48 * 1024 * 1024,
        ),
        name="speaker_rnn_recurrence",
        interpret=interpret,
    )(mcol, xp4, wg, wsg, wsh, wes, bn)

    return jnp.swapaxes(out.reshape(T, B, H), 0, 1)


def kernel(utt_embeds, speaker_ids,
           gW_ih, gW_hh, gb_ih, gb_hh,
           sW_ih, sW_hh, sb_ih, sb_hh,
           eW_ih, eW_hh, eb_ih, eb_hh):
    return _forward(utt_embeds, speaker_ids,
                    gW_ih, gW_hh, gb_ih, gb_hh,
                    sW_ih, sW_hh, sb_ih, sb_hh,
                    eW_ih, eW_hh, eb_ih, eb_hh)
